# Initial kernel scaffold; baseline (speedup 1.0000x reference)
import jax
import jax.numpy as jnp
from jax.experimental import pallas as pl


def kernel(user_weight, item_weight, edge_index):
    raise NotImplementedError("write your pallas kernel here")


# speedup vs baseline: 10.1378x; 10.1378x over previous
import functools

import numpy as np
import jax
import jax.numpy as jnp
from jax import lax
from jax.experimental import pallas as pl
from jax.experimental.pallas import tpu as pltpu
from jax.experimental.pallas import tpu_sc as plsc

NUM_USER = 25000
NUM_ITEM = 25000
DIM = 64
FH = 16
NSPLIT = 4
E = 800000
KEEP = 720000

NCORES = 2
NTILES = 16
NW = NCORES * NTILES

CHUNK = 128
NBLK = 392
EPT = NBLK * CHUNK
E_PAD = NTILES * EPT
PAD = E_PAD - E

PER_SIDE = 25088
N_PAD = 2 * PER_SIDE
NF = NSPLIT * N_PAD
DUMMY = 25024
ROWS_PT = PER_SIDE // NTILES

HTOT = 26624
DPHASES = 8
DPH = HTOT // DPHASES
DS2 = DPH // NTILES


def _tf2x32(k0, k1, x0, x1):
    rots = ((13, 15, 26, 6), (17, 29, 16, 24))
    ks = (k0, k1, np.uint32(k0 ^ k1 ^ np.uint32(0x1BD11BDA)))
    x0 = (x0 + ks[0]).astype(np.uint32)
    x1 = (x1 + ks[1]).astype(np.uint32)

    def rot(x, d):
        return ((x << np.uint32(d)) | (x >> np.uint32(32 - d))).astype(np.uint32)

    for i in range(5):
        for r in rots[i % 2]:
            x0 = (x0 + x1).astype(np.uint32)
            x1 = rot(x1, r) ^ x0
        x0 = (x0 + ks[(i + 1) % 3]).astype(np.uint32)
        x1 = (x1 + ks[(i + 2) % 3] + np.uint32(i + 1)).astype(np.uint32)
    return x0, x1


def _np_random_bits(key, n):
    c1 = np.zeros(n, dtype=np.uint32)
    c2 = np.arange(n, dtype=np.uint32)
    b1, b2 = _tf2x32(key[0], key[1], c1, c2)
    return b1 ^ b2


def _np_split(key):
    c1 = np.zeros(2, dtype=np.uint32)
    c2 = np.arange(2, dtype=np.uint32)
    b1, b2 = _tf2x32(key[0], key[1], c1, c2)
    return np.stack([b1, b2], axis=1)


def _np_permutation(seed, n):
    key = np.array([0, seed], dtype=np.uint32)
    x = np.arange(n, dtype=np.int32)
    num_rounds = int(np.ceil(3 * np.log(max(1, n)) / np.log(2**32 - 1)))
    for _ in range(num_rounds):
        k2 = _np_split(key)
        key, subkey = k2[0], k2[1]
        sort_keys = _np_random_bits(subkey, n)
        order = np.argsort(sort_keys, kind="stable")
        x = x[order]
    return x


@functools.lru_cache(maxsize=1)
def _edge_drop_consts():
    idx1 = _np_permutation(101, E)[:KEEP]
    idx2 = _np_permutation(202, E)[:KEEP]
    m1 = np.zeros(E_PAD, dtype=bool)
    m1[idx1] = True
    m2 = np.zeros(E_PAD, dtype=bool)
    m2[idx2] = True
    ar = np.arange(E_PAD, dtype=np.int32)
    dummy_all = (DUMMY + (ar & 63)).astype(np.int32)
    pad_src = ((ar[:PAD] * 97) % NUM_USER).astype(np.int32)
    return m1, m2, dummy_all, pad_src


def _agg_body(f_hbm, dst_hbm, src_hbm, zeros_hbm, out_hbm,
              dst_v, src_v, rows_a, rows_b, acc, sem_a, sem_b):
    c = lax.axis_index("c")
    s = lax.axis_index("s")
    wid = c * NTILES + s
    base_r = s * ROWS_PT

    pltpu.sync_copy(dst_hbm.at[wid], dst_v)

    for h in range(NSPLIT):
        pltpu.sync_copy(zeros_hbm, rows_a)
        for j in range(ROWS_PT // CHUNK):
            pltpu.sync_copy(rows_a, acc.at[pl.ds(base_r + j * CHUNK, CHUNK)])
        rem = ROWS_PT - (ROWS_PT // CHUNK) * CHUNK
        if rem:
            pltpu.sync_copy(rows_a.at[pl.ds(0, rem)],
                            acc.at[pl.ds(base_r + ROWS_PT - rem, rem)])
        pltpu.sync_copy(src_hbm.at[h * NW + wid], src_v)
        plsc.subcore_barrier()

        pltpu.async_copy(f_hbm.at[src_v.at[0]], rows_a, sem_a)

        def body(jj, carry):
            k0 = 2 * jj
            k1 = k0 + 1
            db = pltpu.async_copy(f_hbm.at[src_v.at[k1]], rows_b, sem_b)
            pltpu.make_async_copy(f_hbm.at[src_v.at[k0]], rows_a, sem_a).wait()
            pltpu.sync_copy(rows_a, acc.at[dst_v.at[k0]], add=True)

            @pl.when(k0 + 2 < NBLK)
            def _():
                pltpu.async_copy(f_hbm.at[src_v.at[k0 + 2]], rows_a, sem_a)

            db.wait()
            pltpu.sync_copy(rows_b, acc.at[dst_v.at[k1]], add=True)
            return carry

        lax.fori_loop(0, NBLK // 2, body, 0)
        plsc.subcore_barrier()

        pltpu.sync_copy(
            acc.at[pl.ds(base_r, ROWS_PT)],
            out_hbm.at[pl.ds(h * N_PAD + c * PER_SIDE + base_r, ROWS_PT)])


def _deg_body(dst_hbm, zeros_hbm, out_hbm, dst_v, hist, red, res, stage):
    c = lax.axis_index("c")
    s = lax.axis_index("s")
    wid = c * NTILES + s

    pltpu.sync_copy(zeros_hbm, hist)
    pltpu.sync_copy(dst_hbm.at[wid], dst_v)

    ones = jnp.full((16,), 1.0, jnp.float32)

    def body(k, carry):
        for t in range(8):
            v = dst_v[k, pl.ds(t * 16, 16)]
            plsc.addupdate_scatter(hist, [v], ones)
        return carry

    lax.fori_loop(0, NBLK, body, 0)

    for h in range(DPHASES):
        pltpu.sync_copy(hist.at[pl.ds(h * DPH, DPH)], stage.at[s])
        plsc.subcore_barrier()
        for t in range(NTILES):
            pltpu.sync_copy(stage.at[t, pl.ds(s * DS2, DS2)], red.at[t])

        def rbody(p, carry):
            acc = red[0, pl.ds(p * 16, 16)]
            for t in range(1, NTILES):
                acc = acc + red[t, pl.ds(p * 16, 16)]
            res[pl.ds(p * 16, 16)] = acc
            return carry

        lax.fori_loop(0, DS2 // 16, rbody, 0)
        pltpu.sync_copy(
            res, out_hbm.at[pl.ds(c * HTOT + h * DPH + s * DS2, DS2)])
        plsc.subcore_barrier()


@functools.lru_cache(maxsize=1)
def _sc_kernels():
    mesh = plsc.VectorSubcoreMesh(core_axis_name="c", subcore_axis_name="s",
                                  num_cores=NCORES, num_subcores=NTILES)
    params = pltpu.CompilerParams(needs_layout_passes=False,
                                  use_tc_tiling_on_sc=False,
                                  has_side_effects=True)
    agg = pl.kernel(
        _agg_body,
        out_type=jax.ShapeDtypeStruct((NF, FH), jnp.float32),
        mesh=mesh,
        compiler_params=params,
        scratch_types=[
            pltpu.VMEM((NBLK, CHUNK), jnp.int32),
            pltpu.VMEM((NBLK, CHUNK), jnp.int32),
            pltpu.VMEM((CHUNK, FH), jnp.float32),
            pltpu.VMEM((CHUNK, FH), jnp.float32),
            pltpu.VMEM_SHARED((PER_SIDE, FH), jnp.float32),
            pltpu.SemaphoreType.DMA,
            pltpu.SemaphoreType.DMA,
        ],
    )
    deg = pl.kernel(
        _deg_body,
        out_type=jax.ShapeDtypeStruct((NCORES * HTOT,), jnp.float32),
        mesh=mesh,
        compiler_params=params,
        scratch_types=[
            pltpu.VMEM((NBLK, CHUNK), jnp.int32),
            pltpu.VMEM((HTOT,), jnp.float32),
            pltpu.VMEM((NTILES, DS2), jnp.float32),
            pltpu.VMEM((DS2,), jnp.float32),
            pltpu.VMEM_SHARED((NTILES, DPH), jnp.float32),
        ],
    )
    return agg, deg


_BR = 6272


def _fspec():
    return pl.BlockSpec((_BR, FH), lambda i: (i, 0))


def _dspec():
    return pl.BlockSpec((_BR, 1), lambda i: (i, 0))


def _scale_pre_call(ego, deg):
    def body(e_ref, d_ref, o_ref):
        d = d_ref[...]
        dp = jnp.where(d == 0.0, 1e-10, d)
        o_ref[...] = e_ref[...] * lax.rsqrt(dp)

    return pl.pallas_call(
        body,
        grid=(NF // _BR,),
        in_specs=[_fspec(), _dspec()],
        out_specs=_fspec(),
        out_shape=jax.ShapeDtypeStruct((NF, FH), jnp.float32),
    )(ego, deg)


def _scale_mid_call(g, deg, gsum):
    def body(g_ref, d_ref, s_ref, f_ref, o_ref):
        d = d_ref[...]
        dp = jnp.where(d == 0.0, 1e-10, d)
        gv = g_ref[...]
        f_ref[...] = gv / dp
        o_ref[...] = s_ref[...] + gv

    return pl.pallas_call(
        body,
        grid=(NF // _BR,),
        in_specs=[_fspec(), _dspec(), _fspec()],
        out_specs=[_fspec(), _fspec()],
        out_shape=[jax.ShapeDtypeStruct((NF, FH), jnp.float32),
                   jax.ShapeDtypeStruct((NF, FH), jnp.float32)],
    )(g, deg, gsum)


def _combine_call(ego, deg, gsum):
    def body(e_ref, d_ref, s_ref, o_ref):
        d = d_ref[...]
        dp = jnp.where(d == 0.0, 1e-10, d)
        o_ref[...] = 0.25 * (e_ref[...] + lax.rsqrt(dp) * s_ref[...])

    return pl.pallas_call(
        body,
        grid=(NF // _BR,),
        in_specs=[_fspec(), _dspec(), _fspec()],
        out_specs=_fspec(),
        out_shape=jax.ShapeDtypeStruct((NF, FH), jnp.float32),
    )(ego, deg, gsum)


def _agg(f2, dst_all, src_all2, zeros32):
    return _sc_kernels()[0](f2, dst_all, src_all2, zeros32)


def _degrees(dst_all, zeros_deg):
    out = _sc_kernels()[1](dst_all, zeros_deg)
    d2 = out.reshape(NCORES, HTOT)[:, :PER_SIDE]
    return jnp.concatenate([d2[0], d2[1]]).reshape(N_PAD, 1)


def _propagate_all(ego2, dst_stack, src_all2, zeros32, zeros_deg):
    def graph_body(_, dst_all):
        deg = _degrees(dst_all, zeros_deg)
        deg2 = jnp.concatenate([deg] * NSPLIT)
        f0 = _scale_pre_call(ego2, deg2)
        gsum0 = jnp.zeros((NF, FH), jnp.float32)

        def layer_body(carry, _x):
            f, gsum = carry
            g = _agg(f, dst_all, src_all2, zeros32)
            f_next, gsum_next = _scale_mid_call(g, deg2, gsum)
            return (f_next, gsum_next), None

        (_, gsum), _ = lax.scan(layer_body, (f0, gsum0), None, length=3)
        out2 = _combine_call(ego2, deg2, gsum)
        return None, out2

    _, outs = lax.scan(graph_body, None, dst_stack)
    res = []
    for gi in range(3):
        out2 = outs[gi]
        user = jnp.concatenate(
            [out2[h * N_PAD:h * N_PAD + NUM_USER] for h in range(NSPLIT)],
            axis=1)
        item = jnp.concatenate(
            [out2[h * N_PAD + PER_SIDE:h * N_PAD + PER_SIDE + NUM_ITEM]
             for h in range(NSPLIT)], axis=1)
        res.append((user, item))
    return res


def kernel(user_weight, item_weight, edge_index):
    m1, m2, dummy_all, pad_src = _edge_drop_consts()
    dummy_all_j = jnp.asarray(dummy_all)

    u = edge_index[:, 0].astype(jnp.int32)
    ig = edge_index[:, 1].astype(jnp.int32)

    dst_u = jnp.concatenate([u, dummy_all_j[E:]])
    dst_i = jnp.concatenate([ig - NUM_USER, dummy_all_j[E:]])
    src_u = jnp.concatenate([ig + (PER_SIDE - NUM_USER), jnp.asarray(pad_src)])
    src_i = jnp.concatenate([u, jnp.asarray(pad_src)])

    src_all = jnp.stack([src_u, src_i]).reshape(NW, NBLK, CHUNK)
    src_all2 = jnp.concatenate(
        [src_all + h * N_PAD for h in range(NSPLIT)], axis=0)

    def dsts(mask):
        if mask is None:
            du, di = dst_u, dst_i
        else:
            mj = jnp.asarray(mask)
            du = jnp.where(mj, dst_u, dummy_all_j)
            di = jnp.where(mj, dst_i, dummy_all_j)
        return jnp.stack([du, di]).reshape(NW, NBLK, CHUNK)

    ego_pad = jnp.zeros((N_PAD, DIM), jnp.float32)
    ego_pad = lax.dynamic_update_slice(ego_pad, user_weight, (0, 0))
    ego_pad = lax.dynamic_update_slice(ego_pad, item_weight, (PER_SIDE, 0))
    ego2 = jnp.concatenate(
        [ego_pad[:, h * FH:(h + 1) * FH] for h in range(NSPLIT)], axis=0)

    zeros32 = jnp.zeros((CHUNK, FH), jnp.float32)
    zeros_deg = jnp.zeros((HTOT,), jnp.float32)

    dst_stack = jnp.stack([dsts(None), dsts(m1), dsts(m2)])
    res = _propagate_all(ego2, dst_stack, src_all2, zeros32, zeros_deg)
    (user_emb, item_emb), (user_s1, item_s1), (user_s2, item_s2) = res
    return (user_emb, item_emb, user_s1, item_s1, user_s2, item_s2)


# speedup vs baseline: 21.5727x; 2.1279x over previous
import functools

import numpy as np
import jax
import jax.numpy as jnp
from jax import lax
from jax.experimental import pallas as pl
from jax.experimental.pallas import tpu as pltpu
from jax.experimental.pallas import tpu_sc as plsc

NUM_USER = 25000
NUM_ITEM = 25000
DIM = 64
FH = 32
NSPLIT = 2
E = 800000
KEEP = 720000

NCORES = 2
NTILES = 16
NW = NCORES * NTILES

CHUNK = 128
NBLK = 392
EPT = NBLK * CHUNK
E_PAD = NTILES * EPT
PAD = E_PAD - E

PER_SIDE = 25088
N_PAD = 2 * PER_SIDE
NF = NSPLIT * N_PAD
DUMMY = 25024
ROWS_PT = PER_SIDE // NTILES

HTOT = 26624
DPHASES = 8
DPH = HTOT // DPHASES
DS2 = DPH // NTILES


def _tf2x32(k0, k1, x0, x1):
    rots = ((13, 15, 26, 6), (17, 29, 16, 24))
    ks = (k0, k1, np.uint32(k0 ^ k1 ^ np.uint32(0x1BD11BDA)))
    x0 = (x0 + ks[0]).astype(np.uint32)
    x1 = (x1 + ks[1]).astype(np.uint32)

    def rot(x, d):
        return ((x << np.uint32(d)) | (x >> np.uint32(32 - d))).astype(np.uint32)

    for i in range(5):
        for r in rots[i % 2]:
            x0 = (x0 + x1).astype(np.uint32)
            x1 = rot(x1, r) ^ x0
        x0 = (x0 + ks[(i + 1) % 3]).astype(np.uint32)
        x1 = (x1 + ks[(i + 2) % 3] + np.uint32(i + 1)).astype(np.uint32)
    return x0, x1


def _np_random_bits(key, n):
    c1 = np.zeros(n, dtype=np.uint32)
    c2 = np.arange(n, dtype=np.uint32)
    b1, b2 = _tf2x32(key[0], key[1], c1, c2)
    return b1 ^ b2


def _np_split(key):
    c1 = np.zeros(2, dtype=np.uint32)
    c2 = np.arange(2, dtype=np.uint32)
    b1, b2 = _tf2x32(key[0], key[1], c1, c2)
    return np.stack([b1, b2], axis=1)


def _np_permutation(seed, n):
    key = np.array([0, seed], dtype=np.uint32)
    x = np.arange(n, dtype=np.int32)
    num_rounds = int(np.ceil(3 * np.log(max(1, n)) / np.log(2**32 - 1)))
    for _ in range(num_rounds):
        k2 = _np_split(key)
        key, subkey = k2[0], k2[1]
        sort_keys = _np_random_bits(subkey, n)
        order = np.argsort(sort_keys, kind="stable")
        x = x[order]
    return x


@functools.lru_cache(maxsize=1)
def _edge_drop_consts():
    idx1 = _np_permutation(101, E)[:KEEP]
    idx2 = _np_permutation(202, E)[:KEEP]
    m1 = np.zeros(E_PAD, dtype=bool)
    m1[idx1] = True
    m2 = np.zeros(E_PAD, dtype=bool)
    m2[idx2] = True
    ar = np.arange(E_PAD, dtype=np.int32)
    dummy_all = (DUMMY + (ar & 63)).astype(np.int32)
    pad_src = ((ar[:PAD] * 97) % NUM_USER).astype(np.int32)
    return m1, m2, dummy_all, pad_src


NRING = 4
WSUB = 224
SB = 28
NSB = NBLK // SB


def _agg_body(f_hbm, dst_hbm, src_hbm, zeros_hbm, out_hbm,
              src0, src1, dst0, dst1, rows, zbuf, acc, sem_g, sem_s, sem_i):
    c = lax.axis_index("c")
    s = lax.axis_index("s")
    wid = c * NTILES + s
    base_r = s * ROWS_PT

    pltpu.sync_copy(zeros_hbm, zbuf)
    for sub in range(ROWS_PT // WSUB):
        pltpu.sync_copy(zbuf, acc.at[pl.ds(base_r + sub * WSUB, WSUB)])

    def process_block(sv, dv):
        for b in range(NRING):
            pltpu.async_copy(f_hbm.at[sv.at[b]], rows.at[b], sem_g)

        def rbody(q, carry):
            k0 = NRING * q
            descs = []
            for b in range(NRING):
                k = k0 + b
                pltpu.make_async_copy(
                    f_hbm.at[sv.at[k]], rows.at[b], sem_g).wait()
                descs.append(pltpu.async_copy(
                    rows.at[b], acc.at[dv.at[k]], sem_s, add=True))
            for b in range(NRING):
                k = k0 + b
                descs[b].wait()

                @pl.when(k + NRING < SB)
                def _():
                    pltpu.async_copy(
                        f_hbm.at[sv.at[k + NRING]], rows.at[b], sem_g)
            return carry

        lax.fori_loop(0, SB // NRING, rbody, 0)

    for h in range(NSPLIT):
        srow = h * NW + wid
        pltpu.sync_copy(dst_hbm.at[wid, pl.ds(0, SB)], dst0)
        pltpu.sync_copy(src_hbm.at[srow, pl.ds(0, SB)], src0)
        plsc.subcore_barrier()

        def sb_pair(t, carry):
            sb0 = 2 * t

            @pl.when(t > 0)
            def _():
                pltpu.make_async_copy(
                    dst_hbm.at[wid, pl.ds(0, SB)], dst0, sem_i).wait()
                pltpu.make_async_copy(
                    src_hbm.at[srow, pl.ds(0, SB)], src0, sem_i).wait()

            pltpu.async_copy(
                dst_hbm.at[wid, pl.ds((sb0 + 1) * SB, SB)], dst1, sem_i)
            pltpu.async_copy(
                src_hbm.at[srow, pl.ds((sb0 + 1) * SB, SB)], src1, sem_i)
            process_block(src0, dst0)
            pltpu.make_async_copy(
                dst_hbm.at[wid, pl.ds(0, SB)], dst1, sem_i).wait()
            pltpu.make_async_copy(
                src_hbm.at[srow, pl.ds(0, SB)], src1, sem_i).wait()

            @pl.when(sb0 + 2 < NSB)
            def _():
                pltpu.async_copy(
                    dst_hbm.at[wid, pl.ds((sb0 + 2) * SB, SB)], dst0, sem_i)
                pltpu.async_copy(
                    src_hbm.at[srow, pl.ds((sb0 + 2) * SB, SB)], src0, sem_i)

            process_block(src1, dst1)
            return carry

        lax.fori_loop(0, NSB // 2, sb_pair, 0)
        plsc.subcore_barrier()

        pltpu.sync_copy(
            acc.at[pl.ds(base_r, ROWS_PT)],
            out_hbm.at[pl.ds(h * N_PAD + c * PER_SIDE + base_r, ROWS_PT)])
        if h + 1 < NSPLIT:
            for sub in range(ROWS_PT // WSUB):
                pltpu.sync_copy(zbuf,
                                acc.at[pl.ds(base_r + sub * WSUB, WSUB)])


def _deg_body(dst_hbm, zeros_hbm, out_hbm, dst_v, hist, red, res, stage):
    c = lax.axis_index("c")
    s = lax.axis_index("s")
    wid = c * NTILES + s

    pltpu.sync_copy(zeros_hbm, hist)
    pltpu.sync_copy(dst_hbm.at[wid], dst_v)

    ones = jnp.full((16,), 1.0, jnp.float32)

    def body(k, carry):
        for t in range(8):
            v = dst_v[k, pl.ds(t * 16, 16)]
            plsc.addupdate_scatter(hist, [v], ones)
        return carry

    lax.fori_loop(0, NBLK, body, 0)

    for h in range(DPHASES):
        pltpu.sync_copy(hist.at[pl.ds(h * DPH, DPH)], stage.at[s])
        plsc.subcore_barrier()
        for t in range(NTILES):
            pltpu.sync_copy(stage.at[t, pl.ds(s * DS2, DS2)], red.at[t])

        def rbody(p, carry):
            acc = red[0, pl.ds(p * 16, 16)]
            for t in range(1, NTILES):
                acc = acc + red[t, pl.ds(p * 16, 16)]
            res[pl.ds(p * 16, 16)] = acc
            return carry

        lax.fori_loop(0, DS2 // 16, rbody, 0)
        pltpu.sync_copy(
            res, out_hbm.at[pl.ds(c * HTOT + h * DPH + s * DS2, DS2)])
        plsc.subcore_barrier()


@functools.lru_cache(maxsize=1)
def _sc_kernels():
    mesh = plsc.VectorSubcoreMesh(core_axis_name="c", subcore_axis_name="s",
                                  num_cores=NCORES, num_subcores=NTILES)
    params = pltpu.CompilerParams(needs_layout_passes=False,
                                  use_tc_tiling_on_sc=False)
    agg = pl.kernel(
        _agg_body,
        out_type=jax.ShapeDtypeStruct((NF, FH), jnp.float32),
        mesh=mesh,
        compiler_params=params,
        scratch_types=[
            pltpu.VMEM((SB, CHUNK), jnp.int32),
            pltpu.VMEM((SB, CHUNK), jnp.int32),
            pltpu.VMEM((SB, CHUNK), jnp.int32),
            pltpu.VMEM((SB, CHUNK), jnp.int32),
            pltpu.VMEM((NRING, CHUNK, FH), jnp.float32),
            pltpu.VMEM((WSUB, FH), jnp.float32),
            pltpu.VMEM_SHARED((PER_SIDE, FH), jnp.float32),
            pltpu.SemaphoreType.DMA,
            pltpu.SemaphoreType.DMA,
            pltpu.SemaphoreType.DMA,
        ],
    )
    deg = pl.kernel(
        _deg_body,
        out_type=jax.ShapeDtypeStruct((NCORES * HTOT,), jnp.float32),
        mesh=mesh,
        compiler_params=params,
        scratch_types=[
            pltpu.VMEM((NBLK, CHUNK), jnp.int32),
            pltpu.VMEM((HTOT,), jnp.float32),
            pltpu.VMEM((NTILES, DS2), jnp.float32),
            pltpu.VMEM((DS2,), jnp.float32),
            pltpu.VMEM_SHARED((NTILES, DPH), jnp.float32),
        ],
    )
    return agg, deg


_BR = 6272


def _fspec():
    return pl.BlockSpec((_BR, FH), lambda i: (i, 0))


def _dspec():
    return pl.BlockSpec((_BR, 1), lambda i: (i, 0))


def _scale_pre_call(ego, deg):
    def body(e_ref, d_ref, o_ref):
        d = d_ref[...]
        dp = jnp.where(d == 0.0, 1e-10, d)
        o_ref[...] = e_ref[...] * lax.rsqrt(dp)

    return pl.pallas_call(
        body,
        grid=(NF // _BR,),
        in_specs=[_fspec(), _dspec()],
        out_specs=_fspec(),
        out_shape=jax.ShapeDtypeStruct((NF, FH), jnp.float32),
    )(ego, deg)


def _scale_mid_call(g, deg, gsum):
    def body(g_ref, d_ref, s_ref, f_ref, o_ref):
        d = d_ref[...]
        dp = jnp.where(d == 0.0, 1e-10, d)
        gv = g_ref[...]
        f_ref[...] = gv / dp
        o_ref[...] = s_ref[...] + gv

    return pl.pallas_call(
        body,
        grid=(NF // _BR,),
        in_specs=[_fspec(), _dspec(), _fspec()],
        out_specs=[_fspec(), _fspec()],
        out_shape=[jax.ShapeDtypeStruct((NF, FH), jnp.float32),
                   jax.ShapeDtypeStruct((NF, FH), jnp.float32)],
    )(g, deg, gsum)


def _combine_call(ego, deg, gsum):
    def body(e_ref, d_ref, s_ref, o_ref):
        d = d_ref[...]
        dp = jnp.where(d == 0.0, 1e-10, d)
        o_ref[...] = 0.25 * (e_ref[...] + lax.rsqrt(dp) * s_ref[...])

    return pl.pallas_call(
        body,
        grid=(NF // _BR,),
        in_specs=[_fspec(), _dspec(), _fspec()],
        out_specs=_fspec(),
        out_shape=jax.ShapeDtypeStruct((NF, FH), jnp.float32),
    )(ego, deg, gsum)


def _agg(f2, dst_all, src_all2, zeros32):
    return _sc_kernels()[0](f2, dst_all, src_all2, zeros32)


def _degsum_call(hists):
    def body(h_ref, o_ref):
        hv = h_ref[...]
        o_ref[...] = jnp.stack(
            [jnp.sum(hv[:NTILES], axis=0), jnp.sum(hv[NTILES:], axis=0)])

    nb = HTOT // 1664
    return pl.pallas_call(
        body,
        grid=(nb,),
        in_specs=[pl.BlockSpec((NW, 1664), lambda j: (0, j))],
        out_specs=pl.BlockSpec((NCORES, 1664), lambda j: (0, j)),
        out_shape=jax.ShapeDtypeStruct((NCORES, HTOT), jnp.float32),
    )(hists)


def _degrees(dst_all, zeros_deg):
    out = _sc_kernels()[1](dst_all, zeros_deg)
    d2 = out.reshape(NCORES, HTOT)[:, :PER_SIDE]
    return jnp.concatenate([d2[0], d2[1]]).reshape(N_PAD, 1)


def _propagate_all(ego2, dst_stack, src_all2, zeros32, zeros_deg):
    def graph_body(tok, dst_all):
        deg = _degrees(dst_all, zeros_deg)
        deg2 = jnp.concatenate([deg] * NSPLIT)
        f0 = _scale_pre_call(ego2, deg2)
        gsum0 = jnp.zeros((NF, FH), jnp.float32)

        def layer_body(carry, _x):
            f, gsum = carry
            g = _agg(f, dst_all, src_all2, zeros32)
            f_next, gsum_next = _scale_mid_call(g, deg2, gsum)
            return (f_next, gsum_next), None

        (_, gsum), _ = lax.scan(layer_body, (f0, gsum0), None, length=3)
        out2 = _combine_call(ego2, deg2, gsum)
        return tok, out2

    tok0 = jnp.zeros((8, 1), jnp.float32)
    _, outs = lax.scan(graph_body, tok0, dst_stack)
    res = []
    for gi in range(3):
        out2 = outs[gi]
        user = jnp.concatenate(
            [out2[h * N_PAD:h * N_PAD + NUM_USER] for h in range(NSPLIT)],
            axis=1)
        item = jnp.concatenate(
            [out2[h * N_PAD + PER_SIDE:h * N_PAD + PER_SIDE + NUM_ITEM]
             for h in range(NSPLIT)], axis=1)
        res.append((user, item))
    return res


def kernel(user_weight, item_weight, edge_index):
    m1, m2, dummy_all, pad_src = _edge_drop_consts()
    dummy_all_j = jnp.asarray(dummy_all)

    u = edge_index[:, 0].astype(jnp.int32)
    ig = edge_index[:, 1].astype(jnp.int32)

    dst_u = jnp.concatenate([u, dummy_all_j[E:]])
    dst_i = jnp.concatenate([ig - NUM_USER, dummy_all_j[E:]])
    src_u = jnp.concatenate([ig + (PER_SIDE - NUM_USER), jnp.asarray(pad_src)])
    src_i = jnp.concatenate([u, jnp.asarray(pad_src)])

    src_all = jnp.stack([src_u, src_i]).reshape(NW, NBLK, CHUNK)
    src_all2 = jnp.concatenate(
        [src_all + h * N_PAD for h in range(NSPLIT)], axis=0)

    def dsts(mask):
        if mask is None:
            du, di = dst_u, dst_i
        else:
            mj = jnp.asarray(mask)
            du = jnp.where(mj, dst_u, dummy_all_j)
            di = jnp.where(mj, dst_i, dummy_all_j)
        return jnp.stack([du, di]).reshape(NW, NBLK, CHUNK)

    ego_pad = jnp.zeros((N_PAD, DIM), jnp.float32)
    ego_pad = lax.dynamic_update_slice(ego_pad, user_weight, (0, 0))
    ego_pad = lax.dynamic_update_slice(ego_pad, item_weight, (PER_SIDE, 0))
    ego2 = jnp.concatenate(
        [ego_pad[:, h * FH:(h + 1) * FH] for h in range(NSPLIT)], axis=0)

    zeros32 = jnp.zeros((WSUB, FH), jnp.float32)
    zeros_deg = jnp.zeros((HTOT,), jnp.float32)

    dst_stack = jnp.stack([dsts(None), dsts(m1), dsts(m2)])
    res = _propagate_all(ego2, dst_stack, src_all2, zeros32, zeros_deg)
    (user_emb, item_emb), (user_s1, item_s1), (user_s2, item_s2) = res
    return (user_emb, item_emb, user_s1, item_s1, user_s2, item_s2)
